# baseline (speedup 1.0000x reference)
import jax
import jax.numpy as jnp
from jax import lax
from jax.experimental import pallas as pl
from jax.experimental.pallas import tpu as pltpu
from jax.experimental.pallas import tpu_sc as plsc

_BATCH = 16384
_D = 32
_NC = 2
_NS = 16
_NW = _NC * _NS
_NUSERS = 1000000
_JPW = 488
_WIN = 4
_WCOLS = 128 * _WIN
_NWINF = _JPW // _WIN
_TAILC = 999936
_CAP = 1280
_CHUNK = 256
_VROWS = _CHUNK + 32
_SLOTS = _NW * _CAP
_DUMP = 2**22
_PPW = _BATCH // _NW


def _sweep_body(uid_hbm, iid_hbm, utab_hbm, itab_hbm, vals_hbm, plist_hbm,
                ids_v, win_v, tail_v, mr_v, mp_v, val_v, pv_v, sem_w, sem_o):
    w = lax.axis_index("s") * _NC + lax.axis_index("c")
    t = lax.rem(w, 2)
    tw = w // 2
    lo = tw * _JPW
    clo = lo * 128
    is_last = tw == 15
    chi = jnp.where(is_last, _NUSERS, clo + _JPW * 128)
    lanes = lax.iota(jnp.int32, 16)

    @pl.when(t == 0)
    def _():
        for c in range(_BATCH // 1024):
            pltpu.sync_copy(uid_hbm.at[pl.ds(c * 1024, 1024)],
                            ids_v.at[pl.ds(c * 1024, 1024)])

    @pl.when(t != 0)
    def _():
        for c in range(_BATCH // 1024):
            pltpu.sync_copy(iid_hbm.at[pl.ds(c * 1024, 1024)],
                            ids_v.at[pl.ds(c * 1024, 1024)])

    dump16 = jnp.full((16,), _DUMP, jnp.int32)

    def initp(k, carry):
        pv_v[pl.ds(k * 16, 16)] = dump16
        return carry

    lax.fori_loop(0, _CAP // 16, initp, 0)

    def match(k, cnt):
        r16 = ids_v[pl.ds(k * 16, 16)]
        m = (r16 >= clo) & (r16 < chi)
        npos = plsc.cumsum(m.astype(jnp.int32))
        slot = jnp.minimum(cnt + npos - 1, _CAP - 1)
        plsc.store_scatter(mr_v, [slot], r16, mask=m)
        plsc.store_scatter(mp_v, [slot], k * 16 + lanes, mask=m)
        return jnp.minimum(cnt + jnp.sum(m.astype(jnp.int32)), _CAP)

    nmatch = lax.fori_loop(0, _BATCH // 16, match, jnp.int32(0))
    ngrp = (nmatch + 15) // 16

    def fire(win, slot):
        cb = pl.multiple_of((lo + win * _WIN) * 128, 128)

        @pl.when(t == 0)
        def _():
            pltpu.async_copy(utab_hbm.at[:, pl.ds(cb, _WCOLS)],
                             win_v.at[slot], sem_w)

        @pl.when(t != 0)
        def _():
            pltpu.async_copy(itab_hbm.at[:, pl.ds(cb, _WCOLS)],
                             win_v.at[slot], sem_w)

    nwin = jnp.where(is_last, _NWINF + 1, _NWINF)
    fire(0, 0)
    wbase = w * _CAP

    def flush_chunk(args):
        cnt, nf = args
        dst = pl.multiple_of(wbase + nf * _CHUNK, _CHUNK)
        pltpu.sync_copy(val_v.at[pl.ds(0, _CHUNK), :],
                        vals_hbm.at[pl.ds(dst, _CHUNK), :])
        for r in range(32):
            for cc in range(8):
                sl = pl.ds(cc * 16, 16)
                val_v[r, sl] = val_v[_CHUNK + r, sl]
        return cnt, nf + 1

    def extract_from(buf, cbl, width, ngrp_, nf):
        def extract(k, cnt):
            valid = (k * 16 + lanes) < nmatch
            r16 = mr_v[pl.ds(k * 16, 16)]
            p16 = mp_v[pl.ds(k * 16, 16)]
            m = valid & (r16 >= cbl) & (r16 < cbl + width)
            npop = jnp.sum(m.astype(jnp.int32))

            def do(cnt):
                col = jnp.where(m, r16 - cbl, 0)
                npos = plsc.cumsum(m.astype(jnp.int32))
                gslot = jnp.minimum(cnt + npos - 1, _CAP - 1)
                lrow = jnp.clip(gslot - nf * _CHUNK, 0, _VROWS - 1)
                plsc.store_scatter(pv_v, [gslot], p16, mask=m)
                for d in range(_D):
                    d16 = jnp.full((16,), d, jnp.int32)
                    v = plsc.load_gather(buf, [d16, col], mask=m)
                    plsc.store_scatter(val_v, [lrow, d16], v, mask=m)
                return jnp.minimum(cnt + npop, _CAP)

            return lax.cond(npop > 0, do, lambda c: c, cnt)

        return extract

    def step(win, carry):
        cnt, nf = carry
        slot = lax.rem(win, 2)

        @pl.when(win + 1 < nwin)
        def _():
            fire(win + 1, lax.rem(win + 1, 2))

        pltpu.make_async_copy(utab_hbm.at[:, pl.ds(0, _WCOLS)],
                              win_v.at[slot], sem_w).wait()
        cbl = (lo + win * _WIN) * 128
        cnt = lax.fori_loop(
            0, ngrp, extract_from(win_v.at[slot], cbl, _WCOLS, ngrp, nf), cnt)
        cnt, nf = lax.cond((cnt - nf * _CHUNK) >= _CHUNK, flush_chunk,
                           lambda a: a, (cnt, nf))
        return cnt, nf

    cnt, nf = lax.fori_loop(0, nwin, step, (jnp.int32(0), jnp.int32(0)))

    @pl.when(is_last & (t == 0))
    def _():
        pltpu.sync_copy(
            utab_hbm.at[:, pl.ds(pl.multiple_of(_TAILC, 128), 128)], tail_v)

    @pl.when(is_last & (t != 0))
    def _():
        pltpu.sync_copy(
            itab_hbm.at[:, pl.ds(pl.multiple_of(_TAILC, 128), 128)], tail_v)

    def tail_step(args):
        cnt, nf = args
        cnt = lax.fori_loop(
            0, ngrp, extract_from(tail_v, _TAILC, 64, ngrp, nf), cnt)
        cnt, nf = lax.cond((cnt - nf * _CHUNK) >= _CHUNK, flush_chunk,
                           lambda a: a, (cnt, nf))
        return cnt, nf

    cnt, nf = lax.cond(is_last, tail_step, lambda a: a, (cnt, nf))

    def final_flush(c, carry):
        cnt, nf = carry
        return lax.cond(nf < _CAP // _CHUNK, flush_chunk, lambda a: a,
                        (cnt, nf))

    cnt, nf = lax.fori_loop(0, _CAP // _CHUNK, final_flush, (cnt, nf))

    pltpu.sync_copy(pv_v, plist_hbm.at[w])


def _dot_body(uid_hbm, iid_hbm, vals_hbm, plist_hbm, ubias_hbm, ibias_hbm,
              out_hbm, pl_v, su_v, si_v, ub_v, ib_v, uidx_v, iidx_v,
              ubv_v, ibv_v, out_v, sem_g, sem_b):
    w = lax.axis_index("s") * _NC + lax.axis_index("c")
    base = w * _PPW
    lanes = lax.iota(jnp.int32, 16)

    for c in range(_SLOTS // 1024):
        pltpu.sync_copy(plist_hbm.at[pl.ds(c * 1024, 1024)],
                        pl_v.at[pl.ds(c * 1024, 1024)])
    for c in range(_PPW // 128):
        pltpu.sync_copy(uid_hbm.at[pl.ds(base + c * 128, 128)],
                        uidx_v.at[c])
        pltpu.sync_copy(iid_hbm.at[pl.ds(base + c * 128, 128)],
                        iidx_v.at[c])

    bias_copies = [
        pltpu.async_copy(ubias_hbm.at[uidx_v.at[c]],
                         ub_v.at[pl.ds(c * 128, 128)], sem_b)
        for c in range(_PPW // 128)
    ] + [
        pltpu.async_copy(ibias_hbm.at[iidx_v.at[c]],
                         ib_v.at[pl.ds(c * 128, 128)], sem_b)
        for c in range(_PPW // 128)
    ]

    for reg in range(_NW):
        regbase = reg * (_CAP // 16)
        tgt = su_v if reg % 2 == 0 else si_v

        def scan_reg(kk, carry, regbase=regbase, tgt=tgt):
            k = regbase + kk
            p16 = pl_v[pl.ds(k * 16, 16)]
            slot16 = k * 16 + lanes
            m = (p16 >= base) & (p16 < base + _PPW)
            loc = jnp.where(m, p16 - base, 0)
            plsc.store_scatter(tgt, [loc // 128, lax.rem(loc, 128)],
                               slot16, mask=m)
            return carry

        lax.fori_loop(0, _CAP // 16, scan_reg, 0)

    for cp in bias_copies:
        cp.wait()

    for c in range(_PPW // 128):
        pltpu.async_copy(vals_hbm.at[su_v.at[c]], ubv_v, sem_g).wait()
        pltpu.async_copy(vals_hbm.at[si_v.at[c]], ibv_v, sem_g).wait()

        def group(g, carry, c=c):
            row = g * 16 + lanes
            sl = pl.ds(c * 128 + g * 16, 16)
            acc = ub_v[sl] + ib_v[sl]
            for d in range(_D):
                d16 = jnp.full((16,), d, jnp.int32)
                u = plsc.load_gather(ubv_v, [row, d16])
                i = plsc.load_gather(ibv_v, [row, d16])
                acc = acc + u * i
            out_v[sl] = acc
            return carry

        lax.fori_loop(0, 8, group, 0)

    pltpu.sync_copy(out_v, out_hbm.at[pl.ds(base, _PPW)])


def kernel(user_ids, item_ids, user_embedding, item_embedding, user_bias,
           item_bias):
    mesh = plsc.VectorSubcoreMesh(core_axis_name="c", subcore_axis_name="s")
    uid = user_ids.astype(jnp.int32)
    iid = item_ids.astype(jnp.int32)

    sweep = pl.kernel(
        _sweep_body,
        out_type=(jax.ShapeDtypeStruct((_SLOTS, 128), jnp.float32),
                  jax.ShapeDtypeStruct((_NW, _CAP), jnp.int32)),
        mesh=mesh,
        compiler_params=pltpu.CompilerParams(needs_layout_passes=False),
        scratch_types=[
            pltpu.VMEM((_BATCH,), jnp.int32),
            pltpu.VMEM((2, 32, _WCOLS), jnp.float32),
            pltpu.VMEM((32, 128), jnp.float32),
            pltpu.VMEM((_CAP,), jnp.int32),
            pltpu.VMEM((_CAP,), jnp.int32),
            pltpu.VMEM((_VROWS, 128), jnp.float32),
            pltpu.VMEM((_CAP,), jnp.int32),
            pltpu.SemaphoreType.DMA,
            pltpu.SemaphoreType.DMA,
        ],
    )
    vals, plist = sweep(uid, iid, user_embedding.T, item_embedding.T)

    dot = pl.kernel(
        _dot_body,
        out_type=jax.ShapeDtypeStruct((_BATCH,), jnp.float32),
        mesh=mesh,
        compiler_params=pltpu.CompilerParams(
            needs_layout_passes=False, use_tc_tiling_on_sc=False),
        scratch_types=[
            pltpu.VMEM((_SLOTS,), jnp.int32),
            pltpu.VMEM((_PPW // 128, 128), jnp.int32),
            pltpu.VMEM((_PPW // 128, 128), jnp.int32),
            pltpu.VMEM((_PPW,), jnp.float32),
            pltpu.VMEM((_PPW,), jnp.float32),
            pltpu.VMEM((_PPW // 128, 128), jnp.int32),
            pltpu.VMEM((_PPW // 128, 128), jnp.int32),
            pltpu.VMEM((128, 128), jnp.float32),
            pltpu.VMEM((128, 128), jnp.float32),
            pltpu.VMEM((_PPW,), jnp.float32),
            pltpu.SemaphoreType.DMA,
            pltpu.SemaphoreType.DMA,
        ],
    )
    return dot(uid, iid, vals, plist.reshape(-1),
               user_bias.reshape(-1), item_bias.reshape(-1))


# speedup vs baseline: 1.8648x; 1.8648x over previous
import jax
import jax.numpy as jnp
from jax import lax
from jax.experimental import pallas as pl
from jax.experimental.pallas import tpu as pltpu
from jax.experimental.pallas import tpu_sc as plsc

_BATCH = 16384
_D = 32
_NC = 2
_NS = 16
_NW = _NC * _NS
_NUSERS = 1000000
_JPW = 488
_WIN = 8
_WCOLS = 128 * _WIN
_NWINF = _JPW // _WIN
_XBLK = 7804
_TAILC = 999936
_CAP = 1280
_WCAP = 128
_CHUNK = 256
_VROWS = _CHUNK + 64
_SLOTS = _NW * _CAP
_DUMP = 2**22
_PPW = _BATCH // _NW


def _popcnt(m):
    return plsc.all_reduce_population_count(m)[0]


def _sweep_body(uid_hbm, iid_hbm, utab_hbm, itab_hbm, vals_hbm, plist_hbm,
                ids_v, win_v, tail_v, mr_v, mp_v, wr_v, wp_v, val_v, pv_v,
                sem_w, sem_o):
    w = lax.axis_index("s") * _NC + lax.axis_index("c")
    t = lax.rem(w, 2)
    tw = w // 2
    lo = tw * _JPW
    clo = lo * 128
    is_last = tw == 15
    chi = jnp.where(is_last, _NUSERS, clo + _JPW * 128)
    lanes = lax.iota(jnp.int32, 16)

    @pl.when(t == 0)
    def _():
        for c in range(_BATCH // 1024):
            pltpu.sync_copy(uid_hbm.at[pl.ds(c * 1024, 1024)],
                            ids_v.at[pl.ds(c * 1024, 1024)])

    @pl.when(t != 0)
    def _():
        for c in range(_BATCH // 1024):
            pltpu.sync_copy(iid_hbm.at[pl.ds(c * 1024, 1024)],
                            ids_v.at[pl.ds(c * 1024, 1024)])

    dump16 = jnp.full((16,), _DUMP, jnp.int32)

    def initp(k, carry):
        pv_v[pl.ds(k * 16, 16)] = dump16
        return carry

    lax.fori_loop(0, _CAP // 16, initp, 0)

    def match(k, cnt):
        r16 = ids_v[pl.ds(k * 16, 16)]
        m = (r16 >= clo) & (r16 < chi)
        npos = plsc.cumsum(m.astype(jnp.int32))
        slot = jnp.minimum(cnt + npos - 1, _CAP - 1)
        plsc.store_scatter(mr_v, [slot], r16, mask=m)
        plsc.store_scatter(mp_v, [slot], k * 16 + lanes, mask=m)
        return jnp.minimum(cnt + _popcnt(m), _CAP)

    nmatch = lax.fori_loop(0, _BATCH // 16, match, jnp.int32(0))
    ngrp = (nmatch + 15) // 16

    def blk_of(win):
        return jnp.where(win >= _NWINF, _XBLK, lo + win * _WIN)

    def fire(win, slot):
        cb = pl.multiple_of(blk_of(win) * 128, 128)

        @pl.when(t == 0)
        def _():
            pltpu.async_copy(utab_hbm.at[:, pl.ds(cb, _WCOLS)],
                             win_v.at[slot], sem_w)

        @pl.when(t != 0)
        def _():
            pltpu.async_copy(itab_hbm.at[:, pl.ds(cb, _WCOLS)],
                             win_v.at[slot], sem_w)

    nwin = jnp.where(is_last, _NWINF + 1, _NWINF)
    fire(0, 0)
    wbase = w * _CAP

    def flush_chunk(args):
        cnt, nf = args
        dst = pl.multiple_of(wbase + nf * _CHUNK, _CHUNK)
        pltpu.sync_copy(val_v.at[pl.ds(0, _CHUNK), :],
                        vals_hbm.at[pl.ds(dst, _CHUNK), :])
        for r in range(_VROWS - _CHUNK):
            for cc in range(8):
                sl = pl.ds(cc * 16, 16)
                val_v[r, sl] = val_v[_CHUNK + r, sl]
        return cnt, nf + 1

    def extract_window(buf, cbl, width, cnt, nf):
        def rescan(k, wcnt):
            valid = (k * 16 + lanes) < nmatch
            r16 = mr_v[pl.ds(k * 16, 16)]
            p16 = mp_v[pl.ds(k * 16, 16)]
            m = valid & (r16 >= cbl) & (r16 < cbl + width)
            npos = plsc.cumsum(m.astype(jnp.int32))
            slot = jnp.minimum(wcnt + npos - 1, _WCAP - 1)
            plsc.store_scatter(wr_v, [slot], r16 - cbl, mask=m)
            plsc.store_scatter(wp_v, [slot], p16, mask=m)
            return jnp.minimum(wcnt + _popcnt(m), _WCAP)

        wcnt = lax.fori_loop(0, ngrp, rescan, jnp.int32(0))
        ngw = (wcnt + 15) // 16

        def extract(k, carry):
            idx = k * 16 + lanes
            m = idx < wcnt
            col = wr_v[pl.ds(k * 16, 16)]
            p16 = wp_v[pl.ds(k * 16, 16)]
            gslot = jnp.minimum(cnt + idx, _CAP - 1)
            lrow = jnp.clip(cnt + idx - nf * _CHUNK, 0, _VROWS - 1)
            plsc.store_scatter(pv_v, [gslot], p16, mask=m)
            for d in range(_D):
                d16 = jnp.full((16,), d, jnp.int32)
                v = plsc.load_gather(buf, [d16, col], mask=m)
                plsc.store_scatter(val_v, [lrow, d16], v, mask=m)
            return carry

        lax.fori_loop(0, ngw, extract, 0)
        return jnp.minimum(cnt + wcnt, _CAP)

    def step(win, carry):
        cnt, nf = carry
        slot = lax.rem(win, 2)

        @pl.when(win + 1 < nwin)
        def _():
            fire(win + 1, lax.rem(win + 1, 2))

        pltpu.make_async_copy(utab_hbm.at[:, pl.ds(0, _WCOLS)],
                              win_v.at[slot], sem_w).wait()
        cbl = blk_of(win) * 128
        cnt = extract_window(win_v.at[slot], cbl, _WCOLS, cnt, nf)
        cnt, nf = lax.cond((cnt - nf * _CHUNK) >= _CHUNK, flush_chunk,
                           lambda a: a, (cnt, nf))
        return cnt, nf

    cnt, nf = lax.fori_loop(0, nwin, step, (jnp.int32(0), jnp.int32(0)))

    @pl.when(is_last & (t == 0))
    def _():
        pltpu.sync_copy(
            utab_hbm.at[:, pl.ds(pl.multiple_of(_TAILC, 128), 128)], tail_v)

    @pl.when(is_last & (t != 0))
    def _():
        pltpu.sync_copy(
            itab_hbm.at[:, pl.ds(pl.multiple_of(_TAILC, 128), 128)], tail_v)

    def tail_step(args):
        cnt, nf = args
        cnt = extract_window(tail_v, _TAILC, 64, cnt, nf)
        cnt, nf = lax.cond((cnt - nf * _CHUNK) >= _CHUNK, flush_chunk,
                           lambda a: a, (cnt, nf))
        return cnt, nf

    cnt, nf = lax.cond(is_last, tail_step, lambda a: a, (cnt, nf))

    def final_flush(c, carry):
        cnt, nf = carry
        return lax.cond(nf < _CAP // _CHUNK, flush_chunk, lambda a: a,
                        (cnt, nf))

    cnt, nf = lax.fori_loop(0, _CAP // _CHUNK, final_flush, (cnt, nf))

    pltpu.sync_copy(pv_v, plist_hbm.at[w])


def _dot_body(uid_hbm, iid_hbm, vals_hbm, plist_hbm, ubias_hbm, ibias_hbm,
              out_hbm, pl_v, su_v, si_v, ub_v, ib_v, uidx_v, iidx_v,
              ubv_v, ibv_v, out_v, sem_g, sem_b):
    w = lax.axis_index("s") * _NC + lax.axis_index("c")
    base = w * _PPW
    lanes = lax.iota(jnp.int32, 16)
    nchunk = _PPW // 128

    for c in range(_SLOTS // 1024):
        pltpu.sync_copy(plist_hbm.at[pl.ds(c * 1024, 1024)],
                        pl_v.at[pl.ds(c * 1024, 1024)])
    for c in range(nchunk):
        pltpu.sync_copy(uid_hbm.at[pl.ds(base + c * 128, 128)],
                        uidx_v.at[c])
        pltpu.sync_copy(iid_hbm.at[pl.ds(base + c * 128, 128)],
                        iidx_v.at[c])

    bias_copies = [
        pltpu.async_copy(ubias_hbm.at[uidx_v.at[c]],
                         ub_v.at[pl.ds(c * 128, 128)], sem_b)
        for c in range(nchunk)
    ] + [
        pltpu.async_copy(ibias_hbm.at[iidx_v.at[c]],
                         ib_v.at[pl.ds(c * 128, 128)], sem_b)
        for c in range(nchunk)
    ]

    for reg in range(_NW):
        regbase = reg * (_CAP // 16)
        tgt = su_v if reg % 2 == 0 else si_v

        def scan_reg(kk, carry, regbase=regbase, tgt=tgt):
            k = regbase + kk
            p16 = pl_v[pl.ds(k * 16, 16)]
            slot16 = k * 16 + lanes
            m = (p16 >= base) & (p16 < base + _PPW)
            loc = jnp.where(m, p16 - base, 0)
            plsc.store_scatter(tgt, [loc // 128, lax.rem(loc, 128)],
                               slot16, mask=m)
            return carry

        lax.fori_loop(0, _CAP // 16, scan_reg, 0)

    for cp in bias_copies:
        cp.wait()

    def fire_gather(c, slot):
        pltpu.async_copy(vals_hbm.at[su_v.at[c]], ubv_v.at[slot], sem_g)
        pltpu.async_copy(vals_hbm.at[si_v.at[c]], ibv_v.at[slot], sem_g)

    def drain_gather(slot):
        pltpu.make_async_copy(vals_hbm.at[pl.ds(0, 128), :],
                              ubv_v.at[slot], sem_g).wait()
        pltpu.make_async_copy(vals_hbm.at[pl.ds(0, 128), :],
                              ibv_v.at[slot], sem_g).wait()

    fire_gather(0, 0)
    for c in range(nchunk):
        slot = c % 2
        if c + 1 < nchunk:
            fire_gather(c + 1, (c + 1) % 2)
        drain_gather(slot)

        def group(g, carry, c=c, slot=slot):
            row = g * 16 + lanes
            sl = pl.ds(c * 128 + g * 16, 16)
            acc = ub_v[sl] + ib_v[sl]
            for d in range(_D):
                d16 = jnp.full((16,), d, jnp.int32)
                u = plsc.load_gather(ubv_v.at[slot], [row, d16])
                i = plsc.load_gather(ibv_v.at[slot], [row, d16])
                acc = acc + u * i
            out_v[sl] = acc
            return carry

        lax.fori_loop(0, 8, group, 0)

    pltpu.sync_copy(out_v, out_hbm.at[pl.ds(base, _PPW)])


def kernel(user_ids, item_ids, user_embedding, item_embedding, user_bias,
           item_bias):
    mesh = plsc.VectorSubcoreMesh(core_axis_name="c", subcore_axis_name="s")
    uid = user_ids.astype(jnp.int32)
    iid = item_ids.astype(jnp.int32)

    sweep = pl.kernel(
        _sweep_body,
        out_type=(jax.ShapeDtypeStruct((_SLOTS, 128), jnp.float32),
                  jax.ShapeDtypeStruct((_NW, _CAP), jnp.int32)),
        mesh=mesh,
        compiler_params=pltpu.CompilerParams(needs_layout_passes=False),
        scratch_types=[
            pltpu.VMEM((_BATCH,), jnp.int32),
            pltpu.VMEM((2, 32, _WCOLS), jnp.float32),
            pltpu.VMEM((32, 128), jnp.float32),
            pltpu.VMEM((_CAP,), jnp.int32),
            pltpu.VMEM((_CAP,), jnp.int32),
            pltpu.VMEM((_WCAP,), jnp.int32),
            pltpu.VMEM((_WCAP,), jnp.int32),
            pltpu.VMEM((_VROWS, 128), jnp.float32),
            pltpu.VMEM((_CAP,), jnp.int32),
            pltpu.SemaphoreType.DMA,
            pltpu.SemaphoreType.DMA,
        ],
    )
    vals, plist = sweep(uid, iid, user_embedding.T, item_embedding.T)

    dot = pl.kernel(
        _dot_body,
        out_type=jax.ShapeDtypeStruct((_BATCH,), jnp.float32),
        mesh=mesh,
        compiler_params=pltpu.CompilerParams(
            needs_layout_passes=False, use_tc_tiling_on_sc=False),
        scratch_types=[
            pltpu.VMEM((_SLOTS,), jnp.int32),
            pltpu.VMEM((_PPW // 128, 128), jnp.int32),
            pltpu.VMEM((_PPW // 128, 128), jnp.int32),
            pltpu.VMEM((_PPW,), jnp.float32),
            pltpu.VMEM((_PPW,), jnp.float32),
            pltpu.VMEM((_PPW // 128, 128), jnp.int32),
            pltpu.VMEM((_PPW // 128, 128), jnp.int32),
            pltpu.VMEM((2, 128, 128), jnp.float32),
            pltpu.VMEM((2, 128, 128), jnp.float32),
            pltpu.VMEM((_PPW,), jnp.float32),
            pltpu.SemaphoreType.DMA,
            pltpu.SemaphoreType.DMA,
        ],
    )
    return dot(uid, iid, vals, plist.reshape(-1),
               user_bias.reshape(-1), item_bias.reshape(-1))


# speedup vs baseline: 2.0269x; 1.0869x over previous
import jax
import jax.numpy as jnp
from jax import lax
from jax.experimental import pallas as pl
from jax.experimental.pallas import tpu as pltpu
from jax.experimental.pallas import tpu_sc as plsc

_BATCH = 16384
_D = 32
_NC = 2
_NS = 16
_NW = _NC * _NS
_NUSERS = 1000000
_JPW = 488
_WIN = 8
_WCOLS = 128 * _WIN
_NWINF = _JPW // _WIN
_XBLK = 7804
_TAILC = 999936
_CAP = 1280
_WCAP = 128
_CHUNK = 256
_VROWS = _CHUNK + 64
_SLOTS = _NW * _CAP
_DUMP = 2**22
_PPW = _BATCH // _NW


def _popcnt(m):
    return plsc.all_reduce_population_count(m)[0]


def _sweep_body(uid_hbm, iid_hbm, utab_hbm, itab_hbm, vals_hbm, plist_hbm,
                ids_v, win_v, tail_v, mr_v, mp_v, wr_v, wp_v, val_v, pv_v,
                sem_w, sem_o):
    w = lax.axis_index("s") * _NC + lax.axis_index("c")
    t = lax.rem(w, 2)
    tw = w // 2
    lo = tw * _JPW
    clo = lo * 128
    is_last = tw == 15
    chi = jnp.where(is_last, _NUSERS, clo + _JPW * 128)
    lanes = lax.iota(jnp.int32, 16)

    @pl.when(t == 0)
    def _():
        for c in range(_BATCH // 1024):
            pltpu.sync_copy(uid_hbm.at[pl.ds(c * 1024, 1024)],
                            ids_v.at[pl.ds(c * 1024, 1024)])

    @pl.when(t != 0)
    def _():
        for c in range(_BATCH // 1024):
            pltpu.sync_copy(iid_hbm.at[pl.ds(c * 1024, 1024)],
                            ids_v.at[pl.ds(c * 1024, 1024)])

    dump16 = jnp.full((16,), _DUMP, jnp.int32)

    def initp(k, carry):
        pv_v[pl.ds(k * 16, 16)] = dump16
        return carry

    lax.fori_loop(0, _CAP // 16, initp, 0)

    def match(k, cnt):
        r16 = ids_v[pl.ds(k * 16, 16)]
        m = (r16 >= clo) & (r16 < chi)
        npos = plsc.cumsum(m.astype(jnp.int32))
        slot = jnp.minimum(cnt + npos - 1, _CAP - 1)
        plsc.store_scatter(mr_v, [slot], r16, mask=m)
        plsc.store_scatter(mp_v, [slot], k * 16 + lanes, mask=m)
        return jnp.minimum(cnt + _popcnt(m), _CAP)

    nmatch = lax.fori_loop(0, _BATCH // 16, match, jnp.int32(0))
    ngrp = (nmatch + 15) // 16

    def blk_of(win):
        return jnp.where(win >= _NWINF, _XBLK, lo + win * _WIN)

    def fire(win, slot):
        cb = pl.multiple_of(blk_of(win) * 128, 128)

        @pl.when(t == 0)
        def _():
            pltpu.async_copy(utab_hbm.at[:, pl.ds(cb, _WCOLS)],
                             win_v.at[slot], sem_w)

        @pl.when(t != 0)
        def _():
            pltpu.async_copy(itab_hbm.at[:, pl.ds(cb, _WCOLS)],
                             win_v.at[slot], sem_w)

    nwin = jnp.where(is_last, _NWINF + 1, _NWINF)
    fire(0, 0)
    wbase = w * _CAP

    def flush_chunk(args):
        cnt, nf = args
        dst = pl.multiple_of(wbase + nf * _CHUNK, _CHUNK)
        pltpu.sync_copy(val_v.at[pl.ds(0, _CHUNK), :],
                        vals_hbm.at[pl.ds(dst, _CHUNK), :])
        for r in range(_VROWS - _CHUNK):
            for cc in range(8):
                sl = pl.ds(cc * 16, 16)
                val_v[r, sl] = val_v[_CHUNK + r, sl]
        return cnt, nf + 1

    def extract_window(buf, cbl, width, cnt, nf):
        def rescan(k, wcnt):
            valid = (k * 16 + lanes) < nmatch
            r16 = mr_v[pl.ds(k * 16, 16)]
            p16 = mp_v[pl.ds(k * 16, 16)]
            m = valid & (r16 >= cbl) & (r16 < cbl + width)
            npos = plsc.cumsum(m.astype(jnp.int32))
            slot = jnp.minimum(wcnt + npos - 1, _WCAP - 1)
            plsc.store_scatter(wr_v, [slot], r16 - cbl, mask=m)
            plsc.store_scatter(wp_v, [slot], p16, mask=m)
            return jnp.minimum(wcnt + _popcnt(m), _WCAP)

        wcnt = lax.fori_loop(0, ngrp, rescan, jnp.int32(0))
        ngw = (wcnt + 15) // 16

        def extract(k, carry):
            idx = k * 16 + lanes
            m = idx < wcnt
            col = wr_v[pl.ds(k * 16, 16)]
            p16 = wp_v[pl.ds(k * 16, 16)]
            gslot = jnp.minimum(cnt + idx, _CAP - 1)
            lrow = jnp.clip(cnt + idx - nf * _CHUNK, 0, _VROWS - 1)
            plsc.store_scatter(pv_v, [gslot], p16, mask=m)
            for d in range(_D):
                d16 = jnp.full((16,), d, jnp.int32)
                v = plsc.load_gather(buf, [d16, col], mask=m)
                plsc.store_scatter(val_v, [lrow, d16], v, mask=m)
            return carry

        lax.fori_loop(0, ngw, extract, 0)
        return jnp.minimum(cnt + wcnt, _CAP)

    def step(win, carry):
        cnt, nf = carry
        slot = lax.rem(win, 2)

        @pl.when(win + 1 < nwin)
        def _():
            fire(win + 1, lax.rem(win + 1, 2))

        pltpu.make_async_copy(utab_hbm.at[:, pl.ds(0, _WCOLS)],
                              win_v.at[slot], sem_w).wait()
        cbl = blk_of(win) * 128
        cnt = extract_window(win_v.at[slot], cbl, _WCOLS, cnt, nf)
        cnt, nf = lax.cond((cnt - nf * _CHUNK) >= _CHUNK, flush_chunk,
                           lambda a: a, (cnt, nf))
        return cnt, nf

    cnt, nf = lax.fori_loop(0, nwin, step, (jnp.int32(0), jnp.int32(0)))

    @pl.when(is_last & (t == 0))
    def _():
        pltpu.sync_copy(
            utab_hbm.at[:, pl.ds(pl.multiple_of(_TAILC, 128), 128)], tail_v)

    @pl.when(is_last & (t != 0))
    def _():
        pltpu.sync_copy(
            itab_hbm.at[:, pl.ds(pl.multiple_of(_TAILC, 128), 128)], tail_v)

    def tail_step(args):
        cnt, nf = args
        cnt = extract_window(tail_v, _TAILC, 64, cnt, nf)
        cnt, nf = lax.cond((cnt - nf * _CHUNK) >= _CHUNK, flush_chunk,
                           lambda a: a, (cnt, nf))
        return cnt, nf

    cnt, nf = lax.cond(is_last, tail_step, lambda a: a, (cnt, nf))

    def final_flush(c, carry):
        cnt, nf = carry
        return lax.cond(nf < _CAP // _CHUNK, flush_chunk, lambda a: a,
                        (cnt, nf))

    cnt, nf = lax.fori_loop(0, _CAP // _CHUNK, final_flush, (cnt, nf))

    pltpu.sync_copy(pv_v, plist_hbm.at[w])


def _dot_body(uid_hbm, iid_hbm, vals_hbm, plist_hbm, ubias_hbm, ibias_hbm,
              out_hbm, pl_v, su_v, si_v, ub_v, ib_v, uidx_v, iidx_v,
              ubv_v, ibv_v, out_v, sem_g, sem_b):
    w = lax.axis_index("s") * _NC + lax.axis_index("c")
    base = w * _PPW
    lanes = lax.iota(jnp.int32, 16)
    nchunk = _PPW // 128

    pltpu.sync_copy(plist_hbm, pl_v)
    for c in range(nchunk):
        pltpu.sync_copy(uid_hbm.at[pl.ds(base + c * 128, 128)],
                        uidx_v.at[c])
        pltpu.sync_copy(iid_hbm.at[pl.ds(base + c * 128, 128)],
                        iidx_v.at[c])

    bias_copies = [
        pltpu.async_copy(ubias_hbm.at[uidx_v.at[c]],
                         ub_v.at[pl.ds(c * 128, 128)], sem_b)
        for c in range(nchunk)
    ] + [
        pltpu.async_copy(ibias_hbm.at[iidx_v.at[c]],
                         ib_v.at[pl.ds(c * 128, 128)], sem_b)
        for c in range(nchunk)
    ]

    for reg in range(_NW):
        regbase = reg * (_CAP // 16)
        tgt = su_v if reg % 2 == 0 else si_v

        def scan_reg(kk, carry, regbase=regbase, tgt=tgt):
            k = regbase + kk
            p16 = pl_v[pl.ds(k * 16, 16)]
            slot16 = k * 16 + lanes
            m = (p16 >= base) & (p16 < base + _PPW)
            loc = jnp.where(m, p16 - base, 0)
            plsc.store_scatter(tgt, [loc // 128, lax.rem(loc, 128)],
                               slot16, mask=m)
            return carry

        lax.fori_loop(0, _CAP // 16, scan_reg, 0)

    for cp in bias_copies:
        cp.wait()

    def fire_gather(c, slot):
        pltpu.async_copy(vals_hbm.at[su_v.at[c]], ubv_v.at[slot], sem_g)
        pltpu.async_copy(vals_hbm.at[si_v.at[c]], ibv_v.at[slot], sem_g)

    def drain_gather(slot):
        pltpu.make_async_copy(vals_hbm.at[pl.ds(0, 128), :],
                              ubv_v.at[slot], sem_g).wait()
        pltpu.make_async_copy(vals_hbm.at[pl.ds(0, 128), :],
                              ibv_v.at[slot], sem_g).wait()

    fire_gather(0, 0)
    for c in range(nchunk):
        slot = c % 2
        if c + 1 < nchunk:
            fire_gather(c + 1, (c + 1) % 2)
        drain_gather(slot)

        def group(g, carry, c=c, slot=slot):
            row = g * 16 + lanes
            sl = pl.ds(c * 128 + g * 16, 16)
            acc = ub_v[sl] + ib_v[sl]
            for d in range(_D):
                d16 = jnp.full((16,), d, jnp.int32)
                u = plsc.load_gather(ubv_v.at[slot], [row, d16])
                i = plsc.load_gather(ibv_v.at[slot], [row, d16])
                acc = acc + u * i
            out_v[sl] = acc
            return carry

        lax.fori_loop(0, 8, group, 0)

    pltpu.sync_copy(out_v, out_hbm.at[pl.ds(base, _PPW)])


def kernel(user_ids, item_ids, user_embedding, item_embedding, user_bias,
           item_bias):
    mesh = plsc.VectorSubcoreMesh(core_axis_name="c", subcore_axis_name="s")
    uid = user_ids.astype(jnp.int32)
    iid = item_ids.astype(jnp.int32)

    sweep = pl.kernel(
        _sweep_body,
        out_type=(jax.ShapeDtypeStruct((_SLOTS, 128), jnp.float32),
                  jax.ShapeDtypeStruct((_NW, _CAP), jnp.int32)),
        mesh=mesh,
        compiler_params=pltpu.CompilerParams(needs_layout_passes=False),
        scratch_types=[
            pltpu.VMEM((_BATCH,), jnp.int32),
            pltpu.VMEM((2, 32, _WCOLS), jnp.float32),
            pltpu.VMEM((32, 128), jnp.float32),
            pltpu.VMEM((_CAP,), jnp.int32),
            pltpu.VMEM((_CAP,), jnp.int32),
            pltpu.VMEM((_WCAP,), jnp.int32),
            pltpu.VMEM((_WCAP,), jnp.int32),
            pltpu.VMEM((_VROWS, 128), jnp.float32),
            pltpu.VMEM((_CAP,), jnp.int32),
            pltpu.SemaphoreType.DMA,
            pltpu.SemaphoreType.DMA,
        ],
    )
    vals, plist = sweep(uid, iid, user_embedding.T, item_embedding.T)

    dot = pl.kernel(
        _dot_body,
        out_type=jax.ShapeDtypeStruct((_BATCH,), jnp.float32),
        mesh=mesh,
        compiler_params=pltpu.CompilerParams(
            needs_layout_passes=False, use_tc_tiling_on_sc=False),
        scratch_types=[
            pltpu.VMEM((_SLOTS,), jnp.int32),
            pltpu.VMEM((_PPW // 128, 128), jnp.int32),
            pltpu.VMEM((_PPW // 128, 128), jnp.int32),
            pltpu.VMEM((_PPW,), jnp.float32),
            pltpu.VMEM((_PPW,), jnp.float32),
            pltpu.VMEM((_PPW // 128, 128), jnp.int32),
            pltpu.VMEM((_PPW // 128, 128), jnp.int32),
            pltpu.VMEM((2, 128, 128), jnp.float32),
            pltpu.VMEM((2, 128, 128), jnp.float32),
            pltpu.VMEM((_PPW,), jnp.float32),
            pltpu.SemaphoreType.DMA,
            pltpu.SemaphoreType.DMA,
        ],
    )
    return dot(uid, iid, vals, plist.reshape(-1),
               user_bias.reshape(-1), item_bias.reshape(-1))


# speedup vs baseline: 2.0557x; 1.0143x over previous
import jax
import jax.numpy as jnp
from jax import lax
from jax.experimental import pallas as pl
from jax.experimental.pallas import tpu as pltpu
from jax.experimental.pallas import tpu_sc as plsc

_BATCH = 16384
_D = 32
_NC = 2
_NS = 16
_NW = _NC * _NS
_NUSERS = 1000000
_JPW = 488
_WIN = 8
_WCOLS = 128 * _WIN
_NWINF = _JPW // _WIN
_XBLK = 7804
_TAILC = 999936
_CAP = 1280
_WCAP = 128
_CHUNK = 256
_VROWS = _CHUNK + 64
_SLOTS = _NW * _CAP
_DUMP = 2**22
_PPW = _BATCH // _NW


def _popcnt(m):
    return plsc.all_reduce_population_count(m)[0]


def _sweep_body(uid_hbm, iid_hbm, utab_hbm, itab_hbm, vals_hbm, plist_hbm,
                ids_v, win_v, tail_v, mr_v, mp_v, wr_v, wp_v, val_v, pv_v,
                sem_w, sem_o):
    w = lax.axis_index("s") * _NC + lax.axis_index("c")
    t = lax.rem(w, 2)
    tw = w // 2
    lo = tw * _JPW
    clo = lo * 128
    is_last = tw == 15
    chi = jnp.where(is_last, _NUSERS, clo + _JPW * 128)
    lanes = lax.iota(jnp.int32, 16)

    @pl.when(t == 0)
    def _():
        for c in range(_BATCH // 1024):
            pltpu.sync_copy(uid_hbm.at[pl.ds(c * 1024, 1024)],
                            ids_v.at[pl.ds(c * 1024, 1024)])

    @pl.when(t != 0)
    def _():
        for c in range(_BATCH // 1024):
            pltpu.sync_copy(iid_hbm.at[pl.ds(c * 1024, 1024)],
                            ids_v.at[pl.ds(c * 1024, 1024)])

    dump16 = jnp.full((16,), _DUMP, jnp.int32)

    def initp(k, carry):
        pv_v[pl.ds(k * 16, 16)] = dump16
        return carry

    lax.fori_loop(0, _CAP // 16, initp, 0)

    def match(k, cnt):
        r16 = ids_v[pl.ds(k * 16, 16)]
        m = (r16 >= clo) & (r16 < chi)
        npos = plsc.cumsum(m.astype(jnp.int32))
        slot = jnp.minimum(cnt + npos - 1, _CAP - 1)
        plsc.store_scatter(mr_v, [slot], r16, mask=m)
        plsc.store_scatter(mp_v, [slot], k * 16 + lanes, mask=m)
        return jnp.minimum(cnt + _popcnt(m), _CAP)

    nmatch = lax.fori_loop(0, _BATCH // 16, match, jnp.int32(0))
    ngrp = (nmatch + 15) // 16

    def blk_of(win):
        return jnp.where(win >= _NWINF, _XBLK, lo + win * _WIN)

    def fire(win, slot):
        cb = pl.multiple_of(blk_of(win) * 128, 128)

        @pl.when(t == 0)
        def _():
            pltpu.async_copy(utab_hbm.at[:, pl.ds(cb, _WCOLS)],
                             win_v.at[slot], sem_w)

        @pl.when(t != 0)
        def _():
            pltpu.async_copy(itab_hbm.at[:, pl.ds(cb, _WCOLS)],
                             win_v.at[slot], sem_w)

    nwin = jnp.where(is_last, _NWINF + 1, _NWINF)
    fire(0, 0)
    wbase = w * _CAP

    def flush_chunk(args):
        cnt, nf = args
        dst = pl.multiple_of(wbase + nf * _CHUNK, _CHUNK)
        pltpu.sync_copy(val_v.at[pl.ds(0, _CHUNK), :],
                        vals_hbm.at[pl.ds(dst, _CHUNK), :])
        for r in range(_VROWS - _CHUNK):
            for cc in range(8):
                sl = pl.ds(cc * 16, 16)
                val_v[r, sl] = val_v[_CHUNK + r, sl]
        return cnt, nf + 1

    def extract_window(buf, cbl, width, cnt, nf):
        def rescan(k, wcnt):
            valid = (k * 16 + lanes) < nmatch
            r16 = mr_v[pl.ds(k * 16, 16)]
            p16 = mp_v[pl.ds(k * 16, 16)]
            m = valid & (r16 >= cbl) & (r16 < cbl + width)
            npos = plsc.cumsum(m.astype(jnp.int32))
            slot = jnp.minimum(wcnt + npos - 1, _WCAP - 1)
            plsc.store_scatter(wr_v, [slot], r16 - cbl, mask=m)
            plsc.store_scatter(wp_v, [slot], p16, mask=m)
            return jnp.minimum(wcnt + _popcnt(m), _WCAP)

        wcnt = lax.fori_loop(0, ngrp, rescan, jnp.int32(0))
        ngw = (wcnt + 15) // 16

        def extract(k, carry):
            idx = k * 16 + lanes
            m = idx < wcnt
            col = wr_v[pl.ds(k * 16, 16)]
            p16 = wp_v[pl.ds(k * 16, 16)]
            gslot = jnp.minimum(cnt + idx, _CAP - 1)
            lrow = jnp.clip(cnt + idx - nf * _CHUNK, 0, _VROWS - 1)
            plsc.store_scatter(pv_v, [gslot], p16, mask=m)
            for d in range(_D):
                d16 = jnp.full((16,), d, jnp.int32)
                v = plsc.load_gather(buf, [d16, col], mask=m)
                plsc.store_scatter(val_v, [lrow, d16], v, mask=m)
            return carry

        lax.fori_loop(0, ngw, extract, 0)
        return jnp.minimum(cnt + wcnt, _CAP)

    def step(win, carry):
        cnt, nf = carry
        slot = lax.rem(win, 2)

        @pl.when(win + 1 < nwin)
        def _():
            fire(win + 1, lax.rem(win + 1, 2))

        pltpu.make_async_copy(utab_hbm.at[:, pl.ds(0, _WCOLS)],
                              win_v.at[slot], sem_w).wait()
        cbl = blk_of(win) * 128
        cnt = extract_window(win_v.at[slot], cbl, _WCOLS, cnt, nf)
        cnt, nf = lax.cond((cnt - nf * _CHUNK) >= _CHUNK, flush_chunk,
                           lambda a: a, (cnt, nf))
        return cnt, nf

    cnt, nf = lax.fori_loop(0, nwin, step, (jnp.int32(0), jnp.int32(0)))

    @pl.when(is_last & (t == 0))
    def _():
        pltpu.sync_copy(
            utab_hbm.at[:, pl.ds(pl.multiple_of(_TAILC, 128), 128)], tail_v)

    @pl.when(is_last & (t != 0))
    def _():
        pltpu.sync_copy(
            itab_hbm.at[:, pl.ds(pl.multiple_of(_TAILC, 128), 128)], tail_v)

    def tail_step(args):
        cnt, nf = args
        cnt = extract_window(tail_v, _TAILC, 64, cnt, nf)
        cnt, nf = lax.cond((cnt - nf * _CHUNK) >= _CHUNK, flush_chunk,
                           lambda a: a, (cnt, nf))
        return cnt, nf

    cnt, nf = lax.cond(is_last, tail_step, lambda a: a, (cnt, nf))

    def final_flush(c, carry):
        cnt, nf = carry
        return lax.cond(nf < _CAP // _CHUNK, flush_chunk, lambda a: a,
                        (cnt, nf))

    cnt, nf = lax.fori_loop(0, _CAP // _CHUNK, final_flush, (cnt, nf))

    pltpu.sync_copy(pv_v, plist_hbm.at[w])


def _dot_body(uid_hbm, iid_hbm, vals_hbm, plist_hbm, ubias_hbm, ibias_hbm,
              out_hbm, pl_v, su_v, si_v, ub_v, ib_v, uidx_v, iidx_v,
              ubv_v, ibv_v, out_v, sem_g, sem_b):
    w = lax.axis_index("s") * _NC + lax.axis_index("c")
    base = w * _PPW
    lanes = lax.iota(jnp.int32, 16)
    nchunk = _PPW // 128

    plc = pltpu.async_copy(plist_hbm, pl_v, sem_g)
    idc = [pltpu.async_copy(uid_hbm.at[pl.ds(base, _PPW)], uidx_v, sem_b),
           pltpu.async_copy(iid_hbm.at[pl.ds(base, _PPW)], iidx_v, sem_b)]
    for cp in idc:
        cp.wait()

    bias_copies = [
        pltpu.async_copy(ubias_hbm.at[uidx_v.at[pl.ds(c * 128, 128)]],
                         ub_v.at[pl.ds(c * 128, 128)], sem_b)
        for c in range(nchunk)
    ] + [
        pltpu.async_copy(ibias_hbm.at[iidx_v.at[pl.ds(c * 128, 128)]],
                         ib_v.at[pl.ds(c * 128, 128)], sem_b)
        for c in range(nchunk)
    ]
    plc.wait()

    for reg in range(_NW):
        regbase = reg * (_CAP // 16)
        tgt = su_v if reg % 2 == 0 else si_v

        def scan_reg(kk, carry, regbase=regbase, tgt=tgt):
            k = regbase + kk
            p16 = pl_v[pl.ds(k * 16, 16)]
            slot16 = k * 16 + lanes
            m = (p16 >= base) & (p16 < base + _PPW)
            loc = jnp.where(m, p16 - base, 0)
            plsc.store_scatter(tgt, [loc // 128, lax.rem(loc, 128)],
                               slot16, mask=m)
            return carry

        lax.fori_loop(0, _CAP // 16, scan_reg, 0)

    for cp in bias_copies:
        cp.wait()

    def fire_gather(c, slot):
        pltpu.async_copy(vals_hbm.at[su_v.at[c]], ubv_v.at[slot], sem_g)
        pltpu.async_copy(vals_hbm.at[si_v.at[c]], ibv_v.at[slot], sem_g)

    def drain_gather(slot):
        pltpu.make_async_copy(vals_hbm.at[pl.ds(0, 128), :],
                              ubv_v.at[slot], sem_g).wait()
        pltpu.make_async_copy(vals_hbm.at[pl.ds(0, 128), :],
                              ibv_v.at[slot], sem_g).wait()

    fire_gather(0, 0)
    for c in range(nchunk):
        slot = c % 2
        if c + 1 < nchunk:
            fire_gather(c + 1, (c + 1) % 2)
        drain_gather(slot)

        def group(g, carry, c=c, slot=slot):
            row = g * 16 + lanes
            sl = pl.ds(c * 128 + g * 16, 16)
            acc = ub_v[sl] + ib_v[sl]
            for d in range(_D):
                d16 = jnp.full((16,), d, jnp.int32)
                u = plsc.load_gather(ubv_v.at[slot], [row, d16])
                i = plsc.load_gather(ibv_v.at[slot], [row, d16])
                acc = acc + u * i
            out_v[sl] = acc
            return carry

        lax.fori_loop(0, 8, group, 0)

    pltpu.sync_copy(out_v, out_hbm.at[pl.ds(base, _PPW)])


def kernel(user_ids, item_ids, user_embedding, item_embedding, user_bias,
           item_bias):
    mesh = plsc.VectorSubcoreMesh(core_axis_name="c", subcore_axis_name="s")
    uid = user_ids.astype(jnp.int32)
    iid = item_ids.astype(jnp.int32)

    sweep = pl.kernel(
        _sweep_body,
        out_type=(jax.ShapeDtypeStruct((_SLOTS, 128), jnp.float32),
                  jax.ShapeDtypeStruct((_NW, _CAP), jnp.int32)),
        mesh=mesh,
        compiler_params=pltpu.CompilerParams(needs_layout_passes=False),
        scratch_types=[
            pltpu.VMEM((_BATCH,), jnp.int32),
            pltpu.VMEM((2, 32, _WCOLS), jnp.float32),
            pltpu.VMEM((32, 128), jnp.float32),
            pltpu.VMEM((_CAP,), jnp.int32),
            pltpu.VMEM((_CAP,), jnp.int32),
            pltpu.VMEM((_WCAP,), jnp.int32),
            pltpu.VMEM((_WCAP,), jnp.int32),
            pltpu.VMEM((_VROWS, 128), jnp.float32),
            pltpu.VMEM((_CAP,), jnp.int32),
            pltpu.SemaphoreType.DMA,
            pltpu.SemaphoreType.DMA,
        ],
    )
    vals, plist = sweep(uid, iid, user_embedding.T, item_embedding.T)

    dot = pl.kernel(
        _dot_body,
        out_type=jax.ShapeDtypeStruct((_BATCH,), jnp.float32),
        mesh=mesh,
        compiler_params=pltpu.CompilerParams(
            needs_layout_passes=False, use_tc_tiling_on_sc=False),
        scratch_types=[
            pltpu.VMEM((_SLOTS,), jnp.int32),
            pltpu.VMEM((_PPW // 128, 128), jnp.int32),
            pltpu.VMEM((_PPW // 128, 128), jnp.int32),
            pltpu.VMEM((_PPW,), jnp.float32),
            pltpu.VMEM((_PPW,), jnp.float32),
            pltpu.VMEM((_PPW,), jnp.int32),
            pltpu.VMEM((_PPW,), jnp.int32),
            pltpu.VMEM((2, 128, 128), jnp.float32),
            pltpu.VMEM((2, 128, 128), jnp.float32),
            pltpu.VMEM((_PPW,), jnp.float32),
            pltpu.SemaphoreType.DMA,
            pltpu.SemaphoreType.DMA,
        ],
    )
    return dot(uid, iid, vals, plist.reshape(-1),
               user_bias.reshape(-1), item_bias.reshape(-1))
